# Initial kernel scaffold; baseline (speedup 1.0000x reference)
import jax
import jax.numpy as jnp
from jax.experimental import pallas as pl


def kernel(h, rel_x, edge_feat, t, edge_index, inner_edge_mask, xk_W1, xk_b1, xk_g, xk_beta, xk_W2, xk_b2, xv_W1, xv_b1, xv_g, xv_beta, xv_W2, xv_b2, xq_W1, xq_b1, xq_g, xq_beta, xq_W2, xq_b2):
    raise NotImplementedError("write your pallas kernel here")


# speedup vs baseline: 23.8196x; 23.8196x over previous
import functools

import jax
import jax.numpy as jnp
import numpy as np
from jax import lax
from jax.experimental import pallas as pl
from jax.experimental.pallas import tpu as pltpu
from jax.experimental.pallas import tpu_sc as plsc

_NC = 2
_NS = 16
_NW = _NC * _NS


def _pick(n, cands):
  for c in cands:
    if n % c == 0:
      return c
  return n


def _ln_relu(y, g, beta):
  mu = jnp.mean(y, -1, keepdims=True)
  var = jnp.mean((y - mu) ** 2, -1, keepdims=True)
  y = (y - mu) * lax.rsqrt(var + 1e-5) * g + beta
  return jnp.maximum(y, 0.0)


def _node_body(h_ref, t_ref, w1_ref, b1_ref, g_ref, be_ref, w2_ref, b2_ref,
               td_ref):
  hb = h_ref[...]
  y = jnp.dot(hb, w1_ref[...], preferred_element_type=jnp.float32) + b1_ref[...]
  y = _ln_relu(y, g_ref[...], be_ref[...])
  q = jnp.dot(y, w2_ref[...], preferred_element_type=jnp.float32) + b2_ref[...]
  td_ref[:, 0:128] = hb
  td_ref[:, 128:256] = q
  td_ref[:, 256:272] = t_ref[...]


def _node_table(h, t, w1, b1, g, be, w2, b2):
  n, d = h.shape
  bn = _pick(n, [1000, 500, 256, 128, 8])
  full = lambda shp: pl.BlockSpec(shp, lambda i: (0, 0))
  return pl.pallas_call(
      _node_body,
      grid=(n // bn,),
      in_specs=[
          pl.BlockSpec((bn, 128), lambda i: (i, 0)),
          pl.BlockSpec((bn, 16), lambda i: (i, 0)),
          full((128, 128)), full((1, 128)), full((1, 128)), full((1, 128)),
          full((128, 128)), full((1, 128)),
      ],
      out_specs=pl.BlockSpec((bn, 272), lambda i: (i, 0)),
      out_shape=jax.ShapeDtypeStruct((n, 272), jnp.float32),
  )(h, t, w1, b1, g, be, w2, b2)


def _gather_tables(td, h, dst, src):
  n = h.shape[0]
  e = dst.shape[0]
  perw = e // _NW
  ch = _pick(perw, [80, 128, 64, 40, 16, 8])
  niter = perw // ch
  mesh = plsc.VectorSubcoreMesh(core_axis_name="c", subcore_axis_name="s", num_cores=_NC, num_subcores=_NS)

  @functools.partial(
      pl.kernel,
      out_type=(jax.ShapeDtypeStruct((e, 272), jnp.float32),
                jax.ShapeDtypeStruct((e, 128), jnp.float32)),
      mesh=mesh,
      compiler_params=pltpu.CompilerParams(use_tc_tiling_on_sc=False),
      scratch_types=[
          pltpu.VMEM((ch,), jnp.int32),
          pltpu.VMEM((ch,), jnp.int32),
          pltpu.VMEM((ch, 272), jnp.float32),
          pltpu.VMEM((ch, 128), jnp.float32),
          pltpu.SemaphoreType.DMA,
          pltpu.SemaphoreType.DMA,
      ],
  )
  def gather_k(td_hbm, h_hbm, dst_hbm, src_hbm, gd_hbm, gs_hbm,
               idxd_v, idxs_v, rowd_v, rows_v, semd, sems):
    wid = lax.axis_index("s") * _NC + lax.axis_index("c")
    base = wid * perw

    @pl.loop(0, niter)
    def _(j):
      off = base + j * ch
      pltpu.sync_copy(dst_hbm.at[pl.ds(off, ch)], idxd_v)
      pltpu.sync_copy(src_hbm.at[pl.ds(off, ch)], idxs_v)
      cpd = pltpu.async_copy(td_hbm.at[idxd_v], rowd_v, semd)
      cps = pltpu.async_copy(h_hbm.at[idxs_v], rows_v, sems)
      cpd.wait()
      cps.wait()
      pltpu.sync_copy(rowd_v, gd_hbm.at[pl.ds(off, ch)])
      pltpu.sync_copy(rows_v, gs_hbm.at[pl.ds(off, ch)])

  return gather_k(td, h, dst, src)


def _edge_body(gd_ref, gs_ref, ef_ref, w1k_ref, b1k_ref, gk_ref, bek_ref,
               w2k_ref, b2k_ref, w1v_ref, b1v_ref, gv_ref, bev_ref, w2v_ref,
               b2v_ref, logit_ref, v_ref, gmax_ref):
  hd = gd_ref[:, 0:128]
  qd = gd_ref[:, 128:256]
  td = gd_ref[:, 256:272]
  hs = gs_ref[...]
  ef = ef_ref[...]
  dot = lambda a, b: jnp.dot(a, b, preferred_element_type=jnp.float32)

  w1k = w1k_ref[...]
  yk = (dot(ef, w1k[0:16]) + dot(hd, w1k[16:144]) + dot(hs, w1k[144:272])
        + b1k_ref[...])
  yk = _ln_relu(yk, gk_ref[...], bek_ref[...])
  k = dot(yk, w2k_ref[...]) + b2k_ref[...]

  w1v = w1v_ref[...]
  yv = (dot(ef, w1v[0:16]) + dot(hd, w1v[16:144]) + dot(hs, w1v[144:272])
        + dot(td, w1v[272:288]) + b1v_ref[...])
  yv = _ln_relu(yv, gv_ref[...], bev_ref[...])
  v_ref[...] = dot(yv, w2v_ref[...]) + b2v_ref[...]

  sel = (lax.broadcasted_iota(jnp.int32, (128, 8), 0) // 16
         == lax.broadcasted_iota(jnp.int32, (128, 8), 1)).astype(jnp.float32)
  logit = dot(qd * k, sel) * 0.25
  logit_ref[...] = logit

  @pl.when(pl.program_id(0) == 0)
  def _():
    gmax_ref[...] = jnp.full((1, 8), -jnp.inf, jnp.float32)

  gmax_ref[...] = jnp.maximum(gmax_ref[...], jnp.max(logit, 0, keepdims=True))


def _edge_mlps(gd, gs, ef, w1k, b1k, gk, bek, w2k, b2k, w1v, b1v, gv, bev,
               w2v, b2v):
  e = gd.shape[0]
  be = _pick(e, [512, 256, 128, 8])
  full = lambda shp: pl.BlockSpec(shp, lambda i: (0, 0))
  return pl.pallas_call(
      _edge_body,
      grid=(e // be,),
      in_specs=[
          pl.BlockSpec((be, 272), lambda i: (i, 0)),
          pl.BlockSpec((be, 128), lambda i: (i, 0)),
          pl.BlockSpec((be, 16), lambda i: (i, 0)),
          full((272, 128)), full((1, 128)), full((1, 128)), full((1, 128)),
          full((128, 128)), full((1, 128)),
          full((288, 128)), full((1, 128)), full((1, 128)), full((1, 128)),
          full((128, 8)), full((1, 8)),
      ],
      out_specs=[
          pl.BlockSpec((be, 8), lambda i: (i, 0)),
          pl.BlockSpec((be, 8), lambda i: (i, 0)),
          pl.BlockSpec((1, 8), lambda i: (0, 0)),
      ],
      out_shape=[
          jax.ShapeDtypeStruct((e, 8), jnp.float32),
          jax.ShapeDtypeStruct((e, 8), jnp.float32),
          jax.ShapeDtypeStruct((1, 8), jnp.float32),
      ],
  )(gd, gs, ef, w1k, b1k, gk, bek, w2k, b2k, w1v, b1v, gv, bev, w2v, b2v)


def _rows_body(l_ref, v_ref, rel_ref, wf_ref, gmax_ref, r_ref):
  ex = jnp.exp(l_ref[...] - gmax_ref[...])
  ev = ex * v_ref[...]
  evw = ev * wf_ref[...]
  evo = ev - evw
  rel = rel_ref[...]
  pieces = [ex]
  for c in range(3):
    pieces.append(evw * rel[:, c:c + 1])
  for c in range(3):
    pieces.append(evo * rel[:, c:c + 1])
  pieces.append(jnp.zeros_like(ex))
  r_ref[...] = jnp.concatenate(pieces, axis=1)


def _build_rows(logits, v, rel, wf, gmax):
  e = logits.shape[0]
  be = _pick(e, [512, 256, 128, 8])
  return pl.pallas_call(
      _rows_body,
      grid=(e // be,),
      in_specs=[
          pl.BlockSpec((be, 8), lambda i: (i, 0)),
          pl.BlockSpec((be, 8), lambda i: (i, 0)),
          pl.BlockSpec((be, 3), lambda i: (i, 0)),
          pl.BlockSpec((be, 1), lambda i: (i, 0)),
          pl.BlockSpec((1, 8), lambda i: (0, 0)),
      ],
      out_specs=pl.BlockSpec((be, 64), lambda i: (i, 0)),
      out_shape=jax.ShapeDtypeStruct((e, 64), jnp.float32),
  )(logits, v, rel, wf, gmax)


def _scatter_rows(r, dst, zeros_tab):
  e = r.shape[0]
  n = zeros_tab.shape[0]
  perw = e // _NW
  ch = _pick(perw, [80, 128, 64, 40, 16, 8])
  niter = perw // ch
  nrows = n // _NS
  mesh = plsc.VectorSubcoreMesh(core_axis_name="c", subcore_axis_name="s", num_cores=_NC, num_subcores=_NS)

  @functools.partial(
      pl.kernel,
      out_type=jax.ShapeDtypeStruct((2, n, 64), jnp.float32),
      mesh=mesh,
      compiler_params=pltpu.CompilerParams(use_tc_tiling_on_sc=False),
      scratch_types=[
          pltpu.VMEM((ch,), jnp.int32),
          pltpu.VMEM((ch, 64), jnp.float32),
          pltpu.VMEM_SHARED((n, 64), jnp.float32),
          pltpu.SemaphoreType.DMA,
      ],
  )
  def scatter_k(r_hbm, dst_hbm, z_hbm, o_hbm, idx_v, row_v, acc_sh, sem):
    c = lax.axis_index("c")
    s = lax.axis_index("s")
    wid = s * _NC + c
    pltpu.sync_copy(z_hbm.at[pl.ds(s * nrows, nrows)],
                    acc_sh.at[pl.ds(s * nrows, nrows)])
    plsc.subcore_barrier()
    base = wid * perw

    @pl.loop(0, niter)
    def _(j):
      off = base + j * ch
      pltpu.sync_copy(dst_hbm.at[pl.ds(off, ch)], idx_v)
      pltpu.sync_copy(r_hbm.at[pl.ds(off, ch)], row_v)
      cp = pltpu.async_copy(row_v, acc_sh.at[idx_v], sem, add=True)
      cp.wait()

    plsc.subcore_barrier()
    pltpu.sync_copy(acc_sh.at[pl.ds(s * nrows, nrows)],
                    o_hbm.at[c, pl.ds(s * nrows, nrows)])

  return scatter_k(r, dst, zeros_tab)


def _combine_body(o_ref, inner_ref, outer_ref):
  a = o_ref[0] + o_ref[1]
  s = a[:, 0:8]
  rinv = jnp.where(s > 0, 1.0 / s, 0.0)
  inner = []
  outer = []
  for c in range(3):
    ui = a[:, 8 + 8 * c:16 + 8 * c]
    uo = a[:, 32 + 8 * c:40 + 8 * c]
    inner.append(jnp.sum(ui * rinv, axis=1, keepdims=True) * 0.125)
    outer.append(jnp.sum(uo * rinv, axis=1, keepdims=True) * 0.125)
  inner_ref[...] = jnp.concatenate(inner, axis=1)
  outer_ref[...] = jnp.concatenate(outer, axis=1)


def _combine(o):
  n = o.shape[1]
  bn = _pick(n, [1000, 500, 256, 128, 8])
  return pl.pallas_call(
      _combine_body,
      grid=(n // bn,),
      in_specs=[pl.BlockSpec((2, bn, 64), lambda i: (0, i, 0))],
      out_specs=[
          pl.BlockSpec((bn, 3), lambda i: (i, 0)),
          pl.BlockSpec((bn, 3), lambda i: (i, 0)),
      ],
      out_shape=[
          jax.ShapeDtypeStruct((n, 3), jnp.float32),
          jax.ShapeDtypeStruct((n, 3), jnp.float32),
      ],
  )(o)


def kernel(h, rel_x, edge_feat, t, edge_index, inner_edge_mask,
           xk_W1, xk_b1, xk_g, xk_beta, xk_W2, xk_b2,
           xv_W1, xv_b1, xv_g, xv_beta, xv_W2, xv_b2,
           xq_W1, xq_b1, xq_g, xq_beta, xq_W2, xq_b2):
  n = h.shape[0]
  e = edge_index.shape[1]
  src = edge_index[0]
  dst = edge_index[1]
  wf = inner_edge_mask.astype(jnp.float32).reshape(e, 1)
  row = lambda x: x.reshape(1, -1)

  td = _node_table(h, t, xq_W1, row(xq_b1), row(xq_g), row(xq_beta), xq_W2,
                   row(xq_b2))
  gd, gs = _gather_tables(td, h, dst, src)
  logits, v, gmax = _edge_mlps(
      gd, gs, edge_feat, xk_W1, row(xk_b1), row(xk_g), row(xk_beta), xk_W2,
      row(xk_b2), xv_W1, row(xv_b1), row(xv_g), row(xv_beta), xv_W2,
      row(xv_b2))
  r = _build_rows(logits, v, rel_x, wf, gmax)
  o = _scatter_rows(r, dst, jnp.zeros((n, 64), jnp.float32))
  inner, outer = _combine(o)
  return (inner, outer)


# speedup vs baseline: 28.0896x; 1.1793x over previous
import functools

import jax
import jax.numpy as jnp
import numpy as np
from jax import lax
from jax.experimental import pallas as pl
from jax.experimental.pallas import tpu as pltpu
from jax.experimental.pallas import tpu_sc as plsc

_NC = 2
_NS = 16
_NW = _NC * _NS


def _pick(n, cands):
  for c in cands:
    if n % c == 0:
      return c
  return n


def _ln_relu(y, g, beta):
  mu = jnp.mean(y, -1, keepdims=True)
  var = jnp.mean((y - mu) ** 2, -1, keepdims=True)
  y = (y - mu) * lax.rsqrt(var + 1e-5) * g + beta
  return jnp.maximum(y, 0.0)


def _node_body(h_ref, t_ref, w1_ref, b1_ref, g_ref, be_ref, w2_ref, b2_ref,
               w1v_ref, td_ref):
  hb = h_ref[...]
  y = jnp.dot(hb, w1_ref[...], preferred_element_type=jnp.float32) + b1_ref[...]
  y = _ln_relu(y, g_ref[...], be_ref[...])
  q = jnp.dot(y, w2_ref[...], preferred_element_type=jnp.float32) + b2_ref[...]
  td_ref[:, 0:128] = hb
  td_ref[:, 128:256] = q
  td_ref[:, 256:384] = jnp.dot(t_ref[...], w1v_ref[272:288],
                               preferred_element_type=jnp.float32)


def _node_table(h, t, w1, b1, g, be, w2, b2, w1v):
  n, d = h.shape
  bn = _pick(n, [1000, 500, 256, 128, 8])
  full = lambda shp: pl.BlockSpec(shp, lambda i: (0, 0))
  return pl.pallas_call(
      _node_body,
      grid=(n // bn,),
      in_specs=[
          pl.BlockSpec((bn, 128), lambda i: (i, 0)),
          pl.BlockSpec((bn, 16), lambda i: (i, 0)),
          full((128, 128)), full((1, 128)), full((1, 128)), full((1, 128)),
          full((128, 128)), full((1, 128)), full((288, 128)),
      ],
      out_specs=pl.BlockSpec((bn, 384), lambda i: (i, 0)),
      out_shape=jax.ShapeDtypeStruct((n, 384), jnp.float32),
  )(h, t, w1, b1, g, be, w2, b2, w1v)


def _gather_tables(td, h, dst, src):
  n = h.shape[0]
  e = dst.shape[0]
  perw = e // _NW
  ch = _pick(perw, [80, 128, 64, 40, 16, 8])
  niter = perw // ch
  mesh = plsc.VectorSubcoreMesh(core_axis_name="c", subcore_axis_name="s", num_cores=_NC, num_subcores=_NS)

  @functools.partial(
      pl.kernel,
      out_type=(jax.ShapeDtypeStruct((e, 384), jnp.float32),
                jax.ShapeDtypeStruct((e, 128), jnp.float32)),
      mesh=mesh,
      scratch_types=[
          pltpu.VMEM((ch,), jnp.int32),
          pltpu.VMEM((ch,), jnp.int32),
          pltpu.VMEM((ch, 384), jnp.float32),
          pltpu.VMEM((ch, 128), jnp.float32),
          pltpu.SemaphoreType.DMA,
          pltpu.SemaphoreType.DMA,
      ],
  )
  def gather_k(td_hbm, h_hbm, dst_hbm, src_hbm, gd_hbm, gs_hbm,
               idxd_v, idxs_v, rowd_v, rows_v, semd, sems):
    wid = lax.axis_index("s") * _NC + lax.axis_index("c")
    base = wid * perw

    @pl.loop(0, niter)
    def _(j):
      off = base + j * ch
      pltpu.sync_copy(dst_hbm.at[pl.ds(off, ch)], idxd_v)
      pltpu.sync_copy(src_hbm.at[pl.ds(off, ch)], idxs_v)
      cpd = pltpu.async_copy(td_hbm.at[idxd_v], rowd_v, semd)
      cps = pltpu.async_copy(h_hbm.at[idxs_v], rows_v, sems)
      cpd.wait()
      cps.wait()
      pltpu.sync_copy(rowd_v, gd_hbm.at[pl.ds(off, ch)])
      pltpu.sync_copy(rows_v, gs_hbm.at[pl.ds(off, ch)])

  return gather_k(td, h, dst, src)


def _edge_body(gd_ref, gs_ref, ef_ref, w1k_ref, b1k_ref, gk_ref, bek_ref,
               w2k_ref, b2k_ref, w1v_ref, b1v_ref, gv_ref, bev_ref, w2v_ref,
               b2v_ref, logit_ref, v_ref, gmax_ref):
  hd = gd_ref[:, 0:128]
  qd = gd_ref[:, 128:256]
  tpd = gd_ref[:, 256:384]
  hs = gs_ref[...]
  ef = ef_ref[...]
  dot = lambda a, b: jnp.dot(a, b, preferred_element_type=jnp.float32)

  w1k = w1k_ref[...]
  yk = (dot(ef, w1k[0:16]) + dot(hd, w1k[16:144]) + dot(hs, w1k[144:272])
        + b1k_ref[...])
  yk = _ln_relu(yk, gk_ref[...], bek_ref[...])
  k = dot(yk, w2k_ref[...]) + b2k_ref[...]

  w1v = w1v_ref[...]
  yv = (dot(ef, w1v[0:16]) + dot(hd, w1v[16:144]) + dot(hs, w1v[144:272])
        + tpd + b1v_ref[...])
  yv = _ln_relu(yv, gv_ref[...], bev_ref[...])
  v_ref[...] = dot(yv, w2v_ref[...]) + b2v_ref[...]

  sel = (lax.broadcasted_iota(jnp.int32, (128, 8), 0) // 16
         == lax.broadcasted_iota(jnp.int32, (128, 8), 1)).astype(jnp.float32)
  logit = dot(qd * k, sel) * 0.25
  logit_ref[...] = logit

  @pl.when(pl.program_id(0) == 0)
  def _():
    gmax_ref[...] = jnp.full((1, 8), -jnp.inf, jnp.float32)

  gmax_ref[...] = jnp.maximum(gmax_ref[...], jnp.max(logit, 0, keepdims=True))


def _edge_mlps(gd, gs, ef, w1k, b1k, gk, bek, w2k, b2k, w1v, b1v, gv, bev,
               w2v, b2v):
  e = gd.shape[0]
  be = _pick(e, [512, 256, 128, 8])
  full = lambda shp: pl.BlockSpec(shp, lambda i: (0, 0))
  return pl.pallas_call(
      _edge_body,
      grid=(e // be,),
      in_specs=[
          pl.BlockSpec((be, 384), lambda i: (i, 0)),
          pl.BlockSpec((be, 128), lambda i: (i, 0)),
          pl.BlockSpec((be, 16), lambda i: (i, 0)),
          full((272, 128)), full((1, 128)), full((1, 128)), full((1, 128)),
          full((128, 128)), full((1, 128)),
          full((288, 128)), full((1, 128)), full((1, 128)), full((1, 128)),
          full((128, 8)), full((1, 8)),
      ],
      out_specs=[
          pl.BlockSpec((be, 8), lambda i: (i, 0)),
          pl.BlockSpec((be, 8), lambda i: (i, 0)),
          pl.BlockSpec((1, 8), lambda i: (0, 0)),
      ],
      out_shape=[
          jax.ShapeDtypeStruct((e, 8), jnp.float32),
          jax.ShapeDtypeStruct((e, 8), jnp.float32),
          jax.ShapeDtypeStruct((1, 8), jnp.float32),
      ],
  )(gd, gs, ef, w1k, b1k, gk, bek, w2k, b2k, w1v, b1v, gv, bev, w2v, b2v)


def _rows_body(l_ref, v_ref, rel_ref, wf_ref, gmax_ref, r_ref):
  ex = jnp.exp(l_ref[...] - gmax_ref[...])
  ev = ex * v_ref[...]
  evw = ev * wf_ref[...]
  evo = ev - evw
  rel = rel_ref[...]
  pieces = [ex]
  for c in range(3):
    pieces.append(evw * rel[:, c:c + 1])
  for c in range(3):
    pieces.append(evo * rel[:, c:c + 1])
  pieces.append(jnp.zeros_like(ex))
  r_ref[...] = jnp.concatenate(pieces, axis=1)


def _build_rows(logits, v, rel, wf, gmax):
  e = logits.shape[0]
  be = _pick(e, [512, 256, 128, 8])
  return pl.pallas_call(
      _rows_body,
      grid=(e // be,),
      in_specs=[
          pl.BlockSpec((be, 8), lambda i: (i, 0)),
          pl.BlockSpec((be, 8), lambda i: (i, 0)),
          pl.BlockSpec((be, 3), lambda i: (i, 0)),
          pl.BlockSpec((be, 1), lambda i: (i, 0)),
          pl.BlockSpec((1, 8), lambda i: (0, 0)),
      ],
      out_specs=pl.BlockSpec((be, 64), lambda i: (i, 0)),
      out_shape=jax.ShapeDtypeStruct((e, 64), jnp.float32),
  )(logits, v, rel, wf, gmax)


def _scatter_rows(r, dst, zeros_tab):
  e = r.shape[0]
  n = zeros_tab.shape[0]
  perw = e // _NW
  ch = _pick(perw, [80, 128, 64, 40, 16, 8])
  niter = perw // ch
  nrows = n // _NS
  mesh = plsc.VectorSubcoreMesh(core_axis_name="c", subcore_axis_name="s", num_cores=_NC, num_subcores=_NS)

  @functools.partial(
      pl.kernel,
      out_type=jax.ShapeDtypeStruct((2, n, 64), jnp.float32),
      mesh=mesh,
      compiler_params=pltpu.CompilerParams(use_tc_tiling_on_sc=False),
      scratch_types=[
          pltpu.VMEM((ch,), jnp.int32),
          pltpu.VMEM((ch, 64), jnp.float32),
          pltpu.VMEM_SHARED((n, 64), jnp.float32),
          pltpu.SemaphoreType.DMA,
      ],
  )
  def scatter_k(r_hbm, dst_hbm, z_hbm, o_hbm, idx_v, row_v, acc_sh, sem):
    c = lax.axis_index("c")
    s = lax.axis_index("s")
    wid = s * _NC + c
    pltpu.sync_copy(z_hbm.at[pl.ds(s * nrows, nrows)],
                    acc_sh.at[pl.ds(s * nrows, nrows)])
    plsc.subcore_barrier()
    base = wid * perw

    @pl.loop(0, niter)
    def _(j):
      off = base + j * ch
      pltpu.sync_copy(dst_hbm.at[pl.ds(off, ch)], idx_v)
      pltpu.sync_copy(r_hbm.at[pl.ds(off, ch)], row_v)
      cp = pltpu.async_copy(row_v, acc_sh.at[idx_v], sem, add=True)
      cp.wait()

    plsc.subcore_barrier()
    pltpu.sync_copy(acc_sh.at[pl.ds(s * nrows, nrows)],
                    o_hbm.at[c, pl.ds(s * nrows, nrows)])

  return scatter_k(r, dst, zeros_tab)


def _combine_body(o_ref, inner_ref, outer_ref):
  a = o_ref[0] + o_ref[1]
  s = a[:, 0:8]
  rinv = jnp.where(s > 0, 1.0 / s, 0.0)
  inner = []
  outer = []
  for c in range(3):
    ui = a[:, 8 + 8 * c:16 + 8 * c]
    uo = a[:, 32 + 8 * c:40 + 8 * c]
    inner.append(jnp.sum(ui * rinv, axis=1, keepdims=True) * 0.125)
    outer.append(jnp.sum(uo * rinv, axis=1, keepdims=True) * 0.125)
  inner_ref[...] = jnp.concatenate(inner, axis=1)
  outer_ref[...] = jnp.concatenate(outer, axis=1)


def _combine(o):
  n = o.shape[1]
  bn = _pick(n, [1000, 500, 256, 128, 8])
  return pl.pallas_call(
      _combine_body,
      grid=(n // bn,),
      in_specs=[pl.BlockSpec((2, bn, 64), lambda i: (0, i, 0))],
      out_specs=[
          pl.BlockSpec((bn, 3), lambda i: (i, 0)),
          pl.BlockSpec((bn, 3), lambda i: (i, 0)),
      ],
      out_shape=[
          jax.ShapeDtypeStruct((n, 3), jnp.float32),
          jax.ShapeDtypeStruct((n, 3), jnp.float32),
      ],
  )(o)


def kernel(h, rel_x, edge_feat, t, edge_index, inner_edge_mask,
           xk_W1, xk_b1, xk_g, xk_beta, xk_W2, xk_b2,
           xv_W1, xv_b1, xv_g, xv_beta, xv_W2, xv_b2,
           xq_W1, xq_b1, xq_g, xq_beta, xq_W2, xq_b2):
  n = h.shape[0]
  e = edge_index.shape[1]
  src = edge_index[0]
  dst = edge_index[1]
  wf = inner_edge_mask.astype(jnp.float32).reshape(e, 1)
  row = lambda x: x.reshape(1, -1)

  td = _node_table(h, t, xq_W1, row(xq_b1), row(xq_g), row(xq_beta), xq_W2,
                   row(xq_b2), xv_W1)
  gd, gs = _gather_tables(td, h, dst, src)
  logits, v, gmax = _edge_mlps(
      gd, gs, edge_feat, xk_W1, row(xk_b1), row(xk_g), row(xk_beta), xk_W2,
      row(xk_b2), xv_W1, row(xv_b1), row(xv_g), row(xv_beta), xv_W2,
      row(xv_b2))
  r = _build_rows(logits, v, rel_x, wf, gmax)
  o = _scatter_rows(r, dst, jnp.zeros((n, 64), jnp.float32))
  inner, outer = _combine(o)
  return (inner, outer)


# speedup vs baseline: 52.8748x; 1.8824x over previous
import functools

import jax
import jax.numpy as jnp
import numpy as np
from jax import lax
from jax.experimental import pallas as pl
from jax.experimental.pallas import tpu as pltpu
from jax.experimental.pallas import tpu_sc as plsc

_NC = 2
_NS = 16
_NW = _NC * _NS


def _pick(n, cands):
  for c in cands:
    if n % c == 0:
      return c
  return n


def _ln_relu(y, g, beta):
  mu = jnp.mean(y, -1, keepdims=True)
  var = jnp.mean((y - mu) ** 2, -1, keepdims=True)
  y = (y - mu) * lax.rsqrt(var + 1e-5) * g + beta
  return jnp.maximum(y, 0.0)


def _node_body(h_ref, t_ref, w1_ref, b1_ref, g_ref, be_ref, w2_ref, b2_ref,
               w1v_ref, td_ref):
  hb = h_ref[...]
  y = jnp.dot(hb, w1_ref[...], preferred_element_type=jnp.float32) + b1_ref[...]
  y = _ln_relu(y, g_ref[...], be_ref[...])
  q = jnp.dot(y, w2_ref[...], preferred_element_type=jnp.float32) + b2_ref[...]
  tp = jnp.dot(t_ref[...], w1v_ref[272:288],
               preferred_element_type=jnp.float32)

  def pack(x):
    lo = lax.bitcast_convert_type(x[:, 0:64].astype(jnp.bfloat16),
                                  jnp.uint16).astype(jnp.uint32)
    hi = lax.bitcast_convert_type(x[:, 64:128].astype(jnp.bfloat16),
                                  jnp.uint16).astype(jnp.uint32)
    return lax.bitcast_convert_type((hi << 16) | lo, jnp.int32)

  td_ref[:, 0:64] = pack(hb)
  td_ref[:, 64:128] = pack(q)
  td_ref[:, 128:192] = pack(tp)
  td_ref[:, 192:256] = jnp.zeros((hb.shape[0], 64), jnp.int32)


def _node_table(h, t, w1, b1, g, be, w2, b2, w1v):
  n, d = h.shape
  bn = _pick(n, [1000, 500, 256, 128, 8])
  full = lambda shp: pl.BlockSpec(shp, lambda i: (0, 0))
  return pl.pallas_call(
      _node_body,
      grid=(n // bn,),
      in_specs=[
          pl.BlockSpec((bn, 128), lambda i: (i, 0)),
          pl.BlockSpec((bn, 16), lambda i: (i, 0)),
          full((128, 128)), full((1, 128)), full((1, 128)), full((1, 128)),
          full((128, 128)), full((1, 128)), full((288, 128)),
      ],
      out_specs=pl.BlockSpec((bn, 256), lambda i: (i, 0)),
      out_shape=jax.ShapeDtypeStruct((n, 256), jnp.int32),
  )(h, t, w1, b1, g, be, w2, b2, w1v)


def _gather_tables(td, h, dst, src, e0, ec):
  n = h.shape[0]
  perw = ec // _NW
  ch = _pick(perw, [80, 128, 64, 40, 16, 8])
  niter = perw // ch
  mesh = plsc.VectorSubcoreMesh(core_axis_name="c", subcore_axis_name="s", num_cores=_NC, num_subcores=_NS)

  @functools.partial(
      pl.kernel,
      out_type=(jax.ShapeDtypeStruct((ec, 256), jnp.int32),
                jax.ShapeDtypeStruct((ec, 128), jnp.float32)),
      mesh=mesh,
      scratch_types=[
          [pltpu.VMEM((ch,), jnp.int32)] * 2,
          [pltpu.VMEM((ch,), jnp.int32)] * 2,
          [pltpu.VMEM((ch, 256), jnp.int32)] * 2,
          [pltpu.VMEM((ch, 128), jnp.float32)] * 2,
          [pltpu.SemaphoreType.DMA] * 2,
          [pltpu.SemaphoreType.DMA] * 2,
      ],
  )
  def gather_k(td_hbm, h_hbm, dst_hbm, src_hbm, gd_hbm, gs_hbm,
               idxd_v, idxs_v, rowd_v, rows_v, semd, sems):
    wid = lax.axis_index("s") * _NC + lax.axis_index("c")
    base = wid * perw

    def start(j, b):
      off = base + j * ch
      pltpu.sync_copy(dst_hbm.at[pl.ds(e0 + off, ch)], idxd_v[b])
      pltpu.sync_copy(src_hbm.at[pl.ds(e0 + off, ch)], idxs_v[b])
      pltpu.async_copy(td_hbm.at[idxd_v[b]], rowd_v[b], semd[b])
      pltpu.async_copy(h_hbm.at[idxs_v[b]], rows_v[b], sems[b])

    def finish(j, b):
      off = base + j * ch
      pltpu.make_async_copy(td_hbm.at[idxd_v[b]], rowd_v[b], semd[b]).wait()
      pltpu.make_async_copy(h_hbm.at[idxs_v[b]], rows_v[b], sems[b]).wait()
      pltpu.sync_copy(rowd_v[b], gd_hbm.at[pl.ds(off, ch)])
      pltpu.sync_copy(rows_v[b], gs_hbm.at[pl.ds(off, ch)])

    start(0, 0)

    @pl.loop(0, niter)
    def _(j):
      @pl.when(j % 2 == 0)
      def _():
        @pl.when(j + 1 < niter)
        def _():
          start(j + 1, 1)
        finish(j, 0)

      @pl.when(j % 2 == 1)
      def _():
        @pl.when(j + 1 < niter)
        def _():
          start(j + 1, 0)
        finish(j, 1)

  return gather_k(td, h, dst, src)


def _edge_body(gd_ref, gs_ref, eft_ref, w1k_ref, b1k_ref, gk_ref, bek_ref,
               w2k_ref, b2k_ref, w1v_ref, b1v_ref, gv_ref, bev_ref, w2v_ref,
               b2vt_ref, lvt_ref, gmax_ref):
  def unpack(p):
    u = lax.bitcast_convert_type(p, jnp.uint32)
    lo = lax.bitcast_convert_type(u << 16, jnp.float32)
    hi = lax.bitcast_convert_type(u & jnp.uint32(0xFFFF0000), jnp.float32)
    return jnp.concatenate([lo, hi], axis=1)

  hd = unpack(gd_ref[:, 0:64])
  qd = unpack(gd_ref[:, 64:128])
  tpd = unpack(gd_ref[:, 128:192])
  hs = gs_ref[...]
  eft = eft_ref[...]
  dot = lambda a, b: jnp.dot(a, b, preferred_element_type=jnp.float32)
  dot_t = lambda a, b: lax.dot_general(
      a, b, (((0,), (0,)), ((), ())), preferred_element_type=jnp.float32)

  w1k = w1k_ref[...]
  yk = (dot_t(eft, w1k[0:16]) + dot(hd, w1k[16:144]) + dot(hs, w1k[144:272])
        + b1k_ref[...])
  yk = _ln_relu(yk, gk_ref[...], bek_ref[...])
  k = dot(yk, w2k_ref[...]) + b2k_ref[...]

  w1v = w1v_ref[...]
  yv = (dot_t(eft, w1v[0:16]) + dot(hd, w1v[16:144]) + dot(hs, w1v[144:272])
        + tpd + b1v_ref[...])
  yv = _ln_relu(yv, gv_ref[...], bev_ref[...])
  vt = lax.dot_general(w2v_ref[...], yv, (((0,), (1,)), ((), ())),
                       preferred_element_type=jnp.float32) + b2vt_ref[...]

  sel = (lax.broadcasted_iota(jnp.int32, (128, 8), 0) // 16
         == lax.broadcasted_iota(jnp.int32, (128, 8), 1)).astype(jnp.float32)
  logit_t = lax.dot_general(sel, qd * k, (((0,), (1,)), ((), ())),
                            preferred_element_type=jnp.float32) * 0.25
  lvt_ref[...] = jnp.concatenate([logit_t, vt], axis=0)

  @pl.when(pl.program_id(0) == 0)
  def _():
    gmax_ref[...] = jnp.full((8, 1), -jnp.inf, jnp.float32)

  gmax_ref[...] = jnp.maximum(gmax_ref[...], jnp.max(logit_t, 1, keepdims=True))


def _edge_mlps(gd, gs, eft, w1k, b1k, gk, bek, w2k, b2k, w1v, b1v, gv, bev,
               w2v, b2vt, i0):
  ec = gd.shape[0]
  be = _pick(ec, [1280, 640, 512, 256, 128, 8])
  full = lambda shp: pl.BlockSpec(shp, lambda i: (0, 0))
  return pl.pallas_call(
      _edge_body,
      grid=(ec // be,),
      in_specs=[
          pl.BlockSpec((be, 256), lambda i: (i, 0)),
          pl.BlockSpec((be, 128), lambda i: (i, 0)),
          pl.BlockSpec((16, be), lambda i: (0, i + i0)),
          full((272, 128)), full((1, 128)), full((1, 128)), full((1, 128)),
          full((128, 128)), full((1, 128)),
          full((288, 128)), full((1, 128)), full((1, 128)), full((1, 128)),
          full((128, 8)), full((8, 1)),
      ],
      out_specs=[
          pl.BlockSpec((16, be), lambda i: (0, i)),
          pl.BlockSpec((8, 1), lambda i: (0, 0)),
      ],
      out_shape=[
          jax.ShapeDtypeStruct((16, ec), jnp.float32),
          jax.ShapeDtypeStruct((8, 1), jnp.float32),
      ],
  )(gd, gs, eft, w1k, b1k, gk, bek, w2k, b2k, w1v, b1v, gv, bev, w2v, b2vt)


def _rows_body(lvt_ref, relt_ref, wft_ref, gmax_ref, rt_ref):
  lvt = lvt_ref[...]
  ex = jnp.exp(lvt[0:8, :] - gmax_ref[...])
  ev = ex * lvt[8:16, :]
  evw = ev * wft_ref[...]
  evo = ev - evw
  relt = relt_ref[...]
  pieces = [ex]
  for c in range(3):
    pieces.append(evw * relt[c:c + 1, :])
  for c in range(3):
    pieces.append(evo * relt[c:c + 1, :])
  pieces.append(jnp.zeros_like(ex))
  rt_ref[...] = jnp.concatenate(pieces, axis=0)


def _build_rows(lvt, relt, wft, gmax, i0):
  ec = lvt.shape[1]
  be = _pick(ec, [2560, 2048, 1024, 640, 512, 256, 128, 8])
  return pl.pallas_call(
      _rows_body,
      grid=(ec // be,),
      in_specs=[
          pl.BlockSpec((16, be), lambda i: (0, i)),
          pl.BlockSpec((3, be), lambda i: (0, i + i0)),
          pl.BlockSpec((1, be), lambda i: (0, i + i0)),
          pl.BlockSpec((8, 1), lambda i: (0, 0)),
      ],
      out_specs=pl.BlockSpec((64, be), lambda i: (0, i)),
      out_shape=jax.ShapeDtypeStruct((64, ec), jnp.float32),
  )(lvt, relt, wft, gmax)


def _scatter_rows(r, dst, zeros_tab, e0):
  ec = r.shape[0]
  n = zeros_tab.shape[0]
  perw = ec // _NW
  ch = _pick(perw, [80, 128, 64, 40, 16, 8])
  niter = perw // ch
  nrows = n // _NS
  mesh = plsc.VectorSubcoreMesh(core_axis_name="c", subcore_axis_name="s", num_cores=_NC, num_subcores=_NS)

  @functools.partial(
      pl.kernel,
      out_type=jax.ShapeDtypeStruct((2, n, 64), jnp.float32),
      mesh=mesh,
      compiler_params=pltpu.CompilerParams(use_tc_tiling_on_sc=False),
      scratch_types=[
          [pltpu.VMEM((ch,), jnp.int32)] * 2,
          [pltpu.VMEM((ch, 64), jnp.float32)] * 2,
          pltpu.VMEM_SHARED((n, 64), jnp.float32),
          [pltpu.SemaphoreType.DMA] * 2,
      ],
  )
  def scatter_k(r_hbm, dst_hbm, z_hbm, o_hbm, idx_v, row_v, acc_sh, sem):
    c = lax.axis_index("c")
    s = lax.axis_index("s")
    wid = s * _NC + c
    pltpu.sync_copy(z_hbm.at[pl.ds(s * nrows, nrows)],
                    acc_sh.at[pl.ds(s * nrows, nrows)])
    plsc.subcore_barrier()
    base = wid * perw

    def load(j, b):
      off = base + j * ch
      pltpu.sync_copy(dst_hbm.at[pl.ds(e0 + off, ch)], idx_v[b])
      pltpu.sync_copy(r_hbm.at[pl.ds(off, ch)], row_v[b])

    def scat(b):
      pltpu.async_copy(row_v[b], acc_sh.at[idx_v[b]], sem[b], add=True)

    def drain(b):
      pltpu.make_async_copy(row_v[b], acc_sh.at[idx_v[b]], sem[b]).wait()

    load(0, 0)
    scat(0)

    @pl.loop(0, niter)
    def _(j):
      @pl.when(j % 2 == 0)
      def _():
        @pl.when(j + 1 < niter)
        def _():
          load(j + 1, 1)
          scat(1)
        drain(0)

      @pl.when(j % 2 == 1)
      def _():
        @pl.when(j + 1 < niter)
        def _():
          load(j + 1, 0)
          scat(0)
        drain(1)

    plsc.subcore_barrier()
    pltpu.sync_copy(acc_sh.at[pl.ds(s * nrows, nrows)],
                    o_hbm.at[c, pl.ds(s * nrows, nrows)])

  return scatter_k(r, dst, zeros_tab)


def _combine_body(o_ref, inner_ref, outer_ref):
  a = jnp.sum(o_ref[...], axis=0)
  s = a[:, 0:8]
  rinv = jnp.where(s > 0, 1.0 / s, 0.0)
  inner = []
  outer = []
  for c in range(3):
    ui = a[:, 8 + 8 * c:16 + 8 * c]
    uo = a[:, 32 + 8 * c:40 + 8 * c]
    inner.append(jnp.sum(ui * rinv, axis=1, keepdims=True) * 0.125)
    outer.append(jnp.sum(uo * rinv, axis=1, keepdims=True) * 0.125)
  inner_ref[...] = jnp.concatenate(inner, axis=1)
  outer_ref[...] = jnp.concatenate(outer, axis=1)


def _combine(o):
  n = o.shape[1]
  nparts = o.shape[0]
  bn = _pick(n, [1000, 500, 256, 128, 8])
  return pl.pallas_call(
      _combine_body,
      grid=(n // bn,),
      in_specs=[pl.BlockSpec((nparts, bn, 64), lambda i: (0, i, 0))],
      out_specs=[
          pl.BlockSpec((bn, 3), lambda i: (i, 0)),
          pl.BlockSpec((bn, 3), lambda i: (i, 0)),
      ],
      out_shape=[
          jax.ShapeDtypeStruct((n, 3), jnp.float32),
          jax.ShapeDtypeStruct((n, 3), jnp.float32),
      ],
  )(o)


def kernel(h, rel_x, edge_feat, t, edge_index, inner_edge_mask,
           xk_W1, xk_b1, xk_g, xk_beta, xk_W2, xk_b2,
           xv_W1, xv_b1, xv_g, xv_beta, xv_W2, xv_b2,
           xq_W1, xq_b1, xq_g, xq_beta, xq_W2, xq_b2):
  n = h.shape[0]
  e = edge_index.shape[1]
  src = edge_index[0]
  dst = edge_index[1]
  wft = inner_edge_mask.astype(jnp.float32).reshape(1, e)
  row = lambda x: x.reshape(1, -1)

  td = _node_table(h, t, xq_W1, row(xq_b1), row(xq_g), row(xq_beta), xq_W2,
                   row(xq_b2), xv_W1)

  grain = 2560
  if e % grain == 0 and e // grain >= 8:
    kg = e // grain
    k0 = max(1, int(kg * 0.19))
    k1 = max(1, int(kg * 0.24))
    k2 = max(1, int(kg * 0.27))
    chunks = [k * grain for k in (k0, k1, k2, kg - k0 - k1 - k2)]
  elif e % grain == 0 and e > grain:
    kg = e // grain
    k0 = (kg + 1) // 2
    chunks = [k0 * grain, (kg - k0) * grain]
  else:
    chunks = [e]
  eft = edge_feat.T
  relt = rel_x.T
  zeros_tab = jnp.zeros((n, 64), jnp.float32)

  lvts, gmaxs, bases = [], [], []
  e0 = 0
  for ec in chunks:
    bases.append(e0)
    gd, gs = _gather_tables(td, h, dst, src, e0, ec)
    be = _pick(ec, [1280, 640, 512, 256, 128, 8])
    lvt, gm = _edge_mlps(
        gd, gs, eft, xk_W1, row(xk_b1), row(xk_g), row(xk_beta), xk_W2,
        row(xk_b2), xv_W1, row(xv_b1), row(xv_g), row(xv_beta), xv_W2,
        xv_b2.reshape(8, 1), e0 // be)
    lvts.append(lvt)
    gmaxs.append(gm)
    e0 += ec
  gmax = gmaxs[0]
  for gm in gmaxs[1:]:
    gmax = jnp.maximum(gmax, gm)

  os = []
  for ci, ec in enumerate(chunks):
    be2 = _pick(ec, [2560, 2048, 1024, 640, 512, 256, 128, 8])
    rt = _build_rows(lvts[ci], relt, wft, gmax, bases[ci] // be2)
    os.append(_scatter_rows(rt.T, dst, zeros_tab, bases[ci]))
  o = os[0] if len(chunks) == 1 else jnp.concatenate(os, axis=0)
  inner, outer = _combine(o)
  return (inner, outer)


# speedup vs baseline: 54.5790x; 1.0322x over previous
import functools

import jax
import jax.numpy as jnp
import numpy as np
from jax import lax
from jax.experimental import pallas as pl
from jax.experimental.pallas import tpu as pltpu
from jax.experimental.pallas import tpu_sc as plsc

_NC = 2
_NS = 16
_NW = _NC * _NS


def _pick(n, cands):
  for c in cands:
    if n % c == 0:
      return c
  return n


def _ln_relu(y, g, beta):
  mu = jnp.mean(y, -1, keepdims=True)
  var = jnp.mean((y - mu) ** 2, -1, keepdims=True)
  y = (y - mu) * lax.rsqrt(var + 1e-5) * g + beta
  return jnp.maximum(y, 0.0)


def _node_body(h_ref, t_ref, w1_ref, b1_ref, g_ref, be_ref, w2_ref, b2_ref,
               w1v_ref, td_ref):
  hb = h_ref[...]
  y = jnp.dot(hb, w1_ref[...], preferred_element_type=jnp.float32) + b1_ref[...]
  y = _ln_relu(y, g_ref[...], be_ref[...])
  q = jnp.dot(y, w2_ref[...], preferred_element_type=jnp.float32) + b2_ref[...]
  td_ref[:, 0:128] = hb
  td_ref[:, 128:256] = q
  td_ref[:, 256:384] = jnp.dot(t_ref[...], w1v_ref[272:288],
                               preferred_element_type=jnp.float32)


def _node_table(h, t, w1, b1, g, be, w2, b2, w1v):
  n, d = h.shape
  bn = _pick(n, [1000, 500, 256, 128, 8])
  full = lambda shp: pl.BlockSpec(shp, lambda i: (0, 0))
  return pl.pallas_call(
      _node_body,
      grid=(n // bn,),
      in_specs=[
          pl.BlockSpec((bn, 128), lambda i: (i, 0)),
          pl.BlockSpec((bn, 16), lambda i: (i, 0)),
          full((128, 128)), full((1, 128)), full((1, 128)), full((1, 128)),
          full((128, 128)), full((1, 128)), full((288, 128)),
      ],
      out_specs=pl.BlockSpec((bn, 384), lambda i: (i, 0)),
      out_shape=jax.ShapeDtypeStruct((n, 384), jnp.float32),
  )(h, t, w1, b1, g, be, w2, b2, w1v)


def _gather_tables(td, h, dst, src, e0, ec):
  n = h.shape[0]
  perw = ec // _NW
  ch = _pick(perw, [80, 128, 64, 40, 16, 8])
  niter = perw // ch
  mesh = plsc.VectorSubcoreMesh(core_axis_name="c", subcore_axis_name="s", num_cores=_NC, num_subcores=_NS)

  @functools.partial(
      pl.kernel,
      out_type=(jax.ShapeDtypeStruct((ec, 384), jnp.float32),
                jax.ShapeDtypeStruct((ec, 128), jnp.float32)),
      mesh=mesh,
      scratch_types=[
          [pltpu.VMEM((ch,), jnp.int32)] * 2,
          [pltpu.VMEM((ch,), jnp.int32)] * 2,
          [pltpu.VMEM((ch, 384), jnp.float32)] * 2,
          [pltpu.VMEM((ch, 128), jnp.float32)] * 2,
          [pltpu.SemaphoreType.DMA] * 2,
          [pltpu.SemaphoreType.DMA] * 2,
      ],
  )
  def gather_k(td_hbm, h_hbm, dst_hbm, src_hbm, gd_hbm, gs_hbm,
               idxd_v, idxs_v, rowd_v, rows_v, semd, sems):
    wid = lax.axis_index("s") * _NC + lax.axis_index("c")
    base = wid * perw

    def start(j, b):
      off = base + j * ch
      pltpu.sync_copy(dst_hbm.at[pl.ds(e0 + off, ch)], idxd_v[b])
      pltpu.sync_copy(src_hbm.at[pl.ds(e0 + off, ch)], idxs_v[b])
      pltpu.async_copy(td_hbm.at[idxd_v[b]], rowd_v[b], semd[b])
      pltpu.async_copy(h_hbm.at[idxs_v[b]], rows_v[b], sems[b])

    def finish(j, b):
      off = base + j * ch
      pltpu.make_async_copy(td_hbm.at[idxd_v[b]], rowd_v[b], semd[b]).wait()
      pltpu.make_async_copy(h_hbm.at[idxs_v[b]], rows_v[b], sems[b]).wait()
      pltpu.sync_copy(rowd_v[b], gd_hbm.at[pl.ds(off, ch)])
      pltpu.sync_copy(rows_v[b], gs_hbm.at[pl.ds(off, ch)])

    start(0, 0)

    @pl.loop(0, niter)
    def _(j):
      @pl.when(j % 2 == 0)
      def _():
        @pl.when(j + 1 < niter)
        def _():
          start(j + 1, 1)
        finish(j, 0)

      @pl.when(j % 2 == 1)
      def _():
        @pl.when(j + 1 < niter)
        def _():
          start(j + 1, 0)
        finish(j, 1)

  return gather_k(td, h, dst, src)


def _edge_body(gd_ref, gs_ref, eft_ref, w1k_ref, b1k_ref, gk_ref, bek_ref,
               w2k_ref, b2k_ref, w1v_ref, b1v_ref, gv_ref, bev_ref, w2v_ref,
               b2vt_ref, lvt_ref, gmax_ref):
  hd = gd_ref[:, 0:128]
  qd = gd_ref[:, 128:256]
  tpd = gd_ref[:, 256:384]
  hs = gs_ref[...]
  eft = eft_ref[...]
  dot = lambda a, b: jnp.dot(a, b, preferred_element_type=jnp.float32)
  dot_t = lambda a, b: lax.dot_general(
      a, b, (((0,), (0,)), ((), ())), preferred_element_type=jnp.float32)

  w1k = w1k_ref[...]
  yk = (dot_t(eft, w1k[0:16]) + dot(hd, w1k[16:144]) + dot(hs, w1k[144:272])
        + b1k_ref[...])
  yk = _ln_relu(yk, gk_ref[...], bek_ref[...])
  k = dot(yk, w2k_ref[...]) + b2k_ref[...]

  w1v = w1v_ref[...]
  yv = (dot_t(eft, w1v[0:16]) + dot(hd, w1v[16:144]) + dot(hs, w1v[144:272])
        + tpd + b1v_ref[...])
  yv = _ln_relu(yv, gv_ref[...], bev_ref[...])
  vt = lax.dot_general(w2v_ref[...], yv, (((0,), (1,)), ((), ())),
                       preferred_element_type=jnp.float32) + b2vt_ref[...]

  sel = (lax.broadcasted_iota(jnp.int32, (128, 8), 0) // 16
         == lax.broadcasted_iota(jnp.int32, (128, 8), 1)).astype(jnp.float32)
  logit_t = lax.dot_general(sel, qd * k, (((0,), (1,)), ((), ())),
                            preferred_element_type=jnp.float32) * 0.25
  lvt_ref[...] = jnp.concatenate([logit_t, vt], axis=0)

  @pl.when(pl.program_id(0) == 0)
  def _():
    gmax_ref[...] = jnp.full((8, 1), -jnp.inf, jnp.float32)

  gmax_ref[...] = jnp.maximum(gmax_ref[...], jnp.max(logit_t, 1, keepdims=True))


def _edge_mlps(gd, gs, eft, w1k, b1k, gk, bek, w2k, b2k, w1v, b1v, gv, bev,
               w2v, b2vt, i0):
  ec = gd.shape[0]
  be = _pick(ec, [1280, 640, 512, 256, 128, 8])
  full = lambda shp: pl.BlockSpec(shp, lambda i: (0, 0))
  return pl.pallas_call(
      _edge_body,
      grid=(ec // be,),
      in_specs=[
          pl.BlockSpec((be, 384), lambda i: (i, 0)),
          pl.BlockSpec((be, 128), lambda i: (i, 0)),
          pl.BlockSpec((16, be), lambda i: (0, i + i0)),
          full((272, 128)), full((1, 128)), full((1, 128)), full((1, 128)),
          full((128, 128)), full((1, 128)),
          full((288, 128)), full((1, 128)), full((1, 128)), full((1, 128)),
          full((128, 8)), full((8, 1)),
      ],
      out_specs=[
          pl.BlockSpec((16, be), lambda i: (0, i)),
          pl.BlockSpec((8, 1), lambda i: (0, 0)),
      ],
      out_shape=[
          jax.ShapeDtypeStruct((16, ec), jnp.float32),
          jax.ShapeDtypeStruct((8, 1), jnp.float32),
      ],
  )(gd, gs, eft, w1k, b1k, gk, bek, w2k, b2k, w1v, b1v, gv, bev, w2v, b2vt)


def _rows_body(lvt_ref, relt_ref, wft_ref, gmax_ref, rt_ref):
  lvt = lvt_ref[...]
  ex = jnp.exp(lvt[0:8, :] - gmax_ref[...])
  ev = ex * lvt[8:16, :]
  evw = ev * wft_ref[...]
  evo = ev - evw
  relt = relt_ref[...]
  pieces = [ex]
  for c in range(3):
    pieces.append(evw * relt[c:c + 1, :])
  for c in range(3):
    pieces.append(evo * relt[c:c + 1, :])
  pieces.append(jnp.zeros_like(ex))
  rt_ref[...] = jnp.concatenate(pieces, axis=0)


def _build_rows(lvt, relt, wft, gmax, i0):
  ec = lvt.shape[1]
  be = _pick(ec, [2560, 2048, 1024, 640, 512, 256, 128, 8])
  return pl.pallas_call(
      _rows_body,
      grid=(ec // be,),
      in_specs=[
          pl.BlockSpec((16, be), lambda i: (0, i)),
          pl.BlockSpec((3, be), lambda i: (0, i + i0)),
          pl.BlockSpec((1, be), lambda i: (0, i + i0)),
          pl.BlockSpec((8, 1), lambda i: (0, 0)),
      ],
      out_specs=pl.BlockSpec((64, be), lambda i: (0, i)),
      out_shape=jax.ShapeDtypeStruct((64, ec), jnp.float32),
  )(lvt, relt, wft, gmax)


def _scatter_rows(r, dst, zeros_tab, e0):
  ec = r.shape[0]
  n = zeros_tab.shape[0]
  perw = ec // _NW
  ch = _pick(perw, [80, 128, 64, 40, 16, 8])
  niter = perw // ch
  nrows = n // _NS
  mesh = plsc.VectorSubcoreMesh(core_axis_name="c", subcore_axis_name="s", num_cores=_NC, num_subcores=_NS)

  @functools.partial(
      pl.kernel,
      out_type=jax.ShapeDtypeStruct((2, n, 64), jnp.float32),
      mesh=mesh,
      compiler_params=pltpu.CompilerParams(use_tc_tiling_on_sc=False),
      scratch_types=[
          [pltpu.VMEM((ch,), jnp.int32)] * 2,
          [pltpu.VMEM((ch, 64), jnp.float32)] * 2,
          pltpu.VMEM_SHARED((n, 64), jnp.float32),
          [pltpu.SemaphoreType.DMA] * 2,
      ],
  )
  def scatter_k(r_hbm, dst_hbm, z_hbm, o_hbm, idx_v, row_v, acc_sh, sem):
    c = lax.axis_index("c")
    s = lax.axis_index("s")
    wid = s * _NC + c
    pltpu.sync_copy(z_hbm.at[pl.ds(s * nrows, nrows)],
                    acc_sh.at[pl.ds(s * nrows, nrows)])
    plsc.subcore_barrier()
    base = wid * perw

    def load(j, b):
      off = base + j * ch
      pltpu.sync_copy(dst_hbm.at[pl.ds(e0 + off, ch)], idx_v[b])
      pltpu.sync_copy(r_hbm.at[pl.ds(off, ch)], row_v[b])

    def scat(b):
      pltpu.async_copy(row_v[b], acc_sh.at[idx_v[b]], sem[b], add=True)

    def drain(b):
      pltpu.make_async_copy(row_v[b], acc_sh.at[idx_v[b]], sem[b]).wait()

    load(0, 0)
    scat(0)

    @pl.loop(0, niter)
    def _(j):
      @pl.when(j % 2 == 0)
      def _():
        @pl.when(j + 1 < niter)
        def _():
          load(j + 1, 1)
          scat(1)
        drain(0)

      @pl.when(j % 2 == 1)
      def _():
        @pl.when(j + 1 < niter)
        def _():
          load(j + 1, 0)
          scat(0)
        drain(1)

    plsc.subcore_barrier()
    pltpu.sync_copy(acc_sh.at[pl.ds(s * nrows, nrows)],
                    o_hbm.at[c, pl.ds(s * nrows, nrows)])

  return scatter_k(r, dst, zeros_tab)


def _combine_body(o_ref, inner_ref, outer_ref):
  a = jnp.sum(o_ref[...], axis=0)
  s = a[:, 0:8]
  rinv = jnp.where(s > 0, 1.0 / s, 0.0)
  inner = []
  outer = []
  for c in range(3):
    ui = a[:, 8 + 8 * c:16 + 8 * c]
    uo = a[:, 32 + 8 * c:40 + 8 * c]
    inner.append(jnp.sum(ui * rinv, axis=1, keepdims=True) * 0.125)
    outer.append(jnp.sum(uo * rinv, axis=1, keepdims=True) * 0.125)
  inner_ref[...] = jnp.concatenate(inner, axis=1)
  outer_ref[...] = jnp.concatenate(outer, axis=1)


def _combine(o):
  n = o.shape[1]
  nparts = o.shape[0]
  bn = _pick(n, [1000, 500, 256, 128, 8])
  return pl.pallas_call(
      _combine_body,
      grid=(n // bn,),
      in_specs=[pl.BlockSpec((nparts, bn, 64), lambda i: (0, i, 0))],
      out_specs=[
          pl.BlockSpec((bn, 3), lambda i: (i, 0)),
          pl.BlockSpec((bn, 3), lambda i: (i, 0)),
      ],
      out_shape=[
          jax.ShapeDtypeStruct((n, 3), jnp.float32),
          jax.ShapeDtypeStruct((n, 3), jnp.float32),
      ],
  )(o)


def kernel(h, rel_x, edge_feat, t, edge_index, inner_edge_mask,
           xk_W1, xk_b1, xk_g, xk_beta, xk_W2, xk_b2,
           xv_W1, xv_b1, xv_g, xv_beta, xv_W2, xv_b2,
           xq_W1, xq_b1, xq_g, xq_beta, xq_W2, xq_b2):
  n = h.shape[0]
  e = edge_index.shape[1]
  src = edge_index[0]
  dst = edge_index[1]
  wft = inner_edge_mask.astype(jnp.float32).reshape(1, e)
  row = lambda x: x.reshape(1, -1)

  td = _node_table(h, t, xq_W1, row(xq_b1), row(xq_g), row(xq_beta), xq_W2,
                   row(xq_b2), xv_W1)

  grain = 2560
  if e % grain == 0 and e // grain >= 8:
    kg = e // grain
    k0 = max(1, int(kg * 0.19))
    k1 = max(1, int(kg * 0.24))
    k2 = max(1, int(kg * 0.27))
    chunks = [k * grain for k in (k0, k1, k2, kg - k0 - k1 - k2)]
  elif e % grain == 0 and e > grain:
    kg = e // grain
    k0 = (kg + 1) // 2
    chunks = [k0 * grain, (kg - k0) * grain]
  else:
    chunks = [e]
  eft = edge_feat.T
  relt = rel_x.T
  zeros_tab = jnp.zeros((n, 64), jnp.float32)

  lvts, gmaxs, bases = [], [], []
  e0 = 0
  for ec in chunks:
    bases.append(e0)
    gd, gs = _gather_tables(td, h, dst, src, e0, ec)
    be = _pick(ec, [1280, 640, 512, 256, 128, 8])
    lvt, gm = _edge_mlps(
        gd, gs, eft, xk_W1, row(xk_b1), row(xk_g), row(xk_beta), xk_W2,
        row(xk_b2), xv_W1, row(xv_b1), row(xv_g), row(xv_beta), xv_W2,
        xv_b2.reshape(8, 1), e0 // be)
    lvts.append(lvt)
    gmaxs.append(gm)
    e0 += ec
  gmax = gmaxs[0]
  for gm in gmaxs[1:]:
    gmax = jnp.maximum(gmax, gm)

  os = []
  for ci, ec in enumerate(chunks):
    be2 = _pick(ec, [2560, 2048, 1024, 640, 512, 256, 128, 8])
    rt = _build_rows(lvts[ci], relt, wft, gmax, bases[ci] // be2)
    os.append(_scatter_rows(rt.T, dst, zeros_tab, bases[ci]))
  o = os[0] if len(chunks) == 1 else jnp.concatenate(os, axis=0)
  inner, outer = _combine(o)
  return (inner, outer)
